# speedup vs baseline: 2.1129x; 2.1129x over previous
import jax
import jax.numpy as jnp
from jax import lax
from jax.experimental import pallas as pl
from jax.experimental.pallas import tpu as pltpu
from jax.experimental.pallas import tpu_sc as plsc

M = 400000
C = 128
R = 128
NUM_CHUNKS = M // R
NC = 2
NS = 16
NW = NC * NS
ITERS = -(-NUM_CHUNKS // NW)


def _sc_body(x_hbm, up_hbm, idx_hbm, out_hbm, idx_v, gath_v, up_v, sem):
    wid = lax.axis_index("s") * NC + lax.axis_index("c")

    def chunk_step(i, carry):
        chunk = wid + i * NW

        @pl.when(chunk < NUM_CHUNKS)
        def _():
            base = chunk * R
            pltpu.sync_copy(idx_hbm.at[pl.ds(base, R)], idx_v)
            gather = pltpu.async_copy(x_hbm.at[idx_v], gath_v, sem)
            pltpu.sync_copy(up_hbm.at[pl.ds(base, R)], up_v)
            gather.wait()

            def add_row(r, c2):
                for l in range(C // 16):
                    s = pl.ds(l * 16, 16)
                    up_v[r, s] = up_v[r, s] + gath_v[r, s]
                return c2

            lax.fori_loop(0, R, add_row, 0)
            pltpu.sync_copy(up_v, out_hbm.at[pl.ds(base, R)])

        return carry

    lax.fori_loop(0, ITERS, chunk_step, 0)


def kernel(x_features, up_x_features, unq_inv):
    idx = unq_inv.astype(jnp.int32)
    mesh = plsc.VectorSubcoreMesh(
        core_axis_name="c", subcore_axis_name="s",
        num_cores=NC, num_subcores=NS)
    f = pl.kernel(
        _sc_body,
        out_type=jax.ShapeDtypeStruct((M, C), jnp.float32),
        mesh=mesh,
        scratch_types=[
            pltpu.VMEM((R,), jnp.int32),
            pltpu.VMEM((R, C), jnp.float32),
            pltpu.VMEM((R, C), jnp.float32),
            pltpu.SemaphoreType.DMA,
        ],
    )
    return f(x_features, up_x_features, idx)


# speedup vs baseline: 2.2337x; 1.0572x over previous
import jax
import jax.numpy as jnp
from jax import lax
from jax.experimental import pallas as pl
from jax.experimental.pallas import tpu as pltpu
from jax.experimental.pallas import tpu_sc as plsc

M = 400000
C = 128
R = 128
NUM_CHUNKS = M // R
NC = 2
NS = 16
NW = NC * NS
ITERS = -(-NUM_CHUNKS // NW)


def _sc_body(x_hbm, up_hbm, idx_hbm, out_hbm, idx_v, up_v, sem):
    wid = lax.axis_index("s") * NC + lax.axis_index("c")

    def chunk_step(i, carry):
        chunk = wid + i * NW

        @pl.when(chunk < NUM_CHUNKS)
        def _():
            base = chunk * R
            pltpu.sync_copy(idx_hbm.at[pl.ds(base, R)], idx_v)
            pltpu.sync_copy(up_hbm.at[pl.ds(base, R)], up_v)
            pltpu.async_copy(x_hbm.at[idx_v], up_v, sem, add=True).wait()
            pltpu.sync_copy(up_v, out_hbm.at[pl.ds(base, R)])

        return carry

    lax.fori_loop(0, ITERS, chunk_step, 0)


def kernel(x_features, up_x_features, unq_inv):
    idx = unq_inv.astype(jnp.int32)
    mesh = plsc.VectorSubcoreMesh(
        core_axis_name="c", subcore_axis_name="s",
        num_cores=NC, num_subcores=NS)
    f = pl.kernel(
        _sc_body,
        out_type=jax.ShapeDtypeStruct((M, C), jnp.float32),
        mesh=mesh,
        scratch_types=[
            pltpu.VMEM((R,), jnp.int32),
            pltpu.VMEM((R, C), jnp.float32),
            pltpu.SemaphoreType.DMA,
        ],
    )
    return f(x_features, up_x_features, idx)


# speedup vs baseline: 3.9135x; 1.7520x over previous
import jax
import jax.numpy as jnp
from jax import lax
from jax.experimental import pallas as pl
from jax.experimental.pallas import tpu as pltpu
from jax.experimental.pallas import tpu_sc as plsc

M = 400000
C = 128
R = 128
NUM_CHUNKS = M // R
NC = 2
NS = 16
NW = NC * NS
ITERS = -(-NUM_CHUNKS // NW)
NBUF = 4


def _sc_body(x_hbm, up_hbm, idx_hbm, out_hbm, idx_v, up_v, lsem, gsem, ssem):
    wid = lax.axis_index("s") * NC + lax.axis_index("c")
    ni = (NUM_CHUNKS - wid + NW - 1) // NW

    def base_of(k):
        return (wid + k * NW) * R

    def ldescs(k, b):
        base = base_of(k)
        return (pltpu.make_async_copy(idx_hbm.at[pl.ds(base, R)],
                                      idx_v.at[b], lsem.at[b]),
                pltpu.make_async_copy(up_hbm.at[pl.ds(base, R)],
                                      up_v.at[b], lsem.at[b]))

    def gdesc(b):
        return pltpu.make_async_copy(x_hbm.at[idx_v.at[b]], up_v.at[b],
                                     gsem.at[b])

    def sdesc(k, b):
        return pltpu.make_async_copy(up_v.at[b],
                                     out_hbm.at[pl.ds(base_of(k), R)],
                                     ssem.at[b])

    d1, d2 = ldescs(0, 0)
    d1.start()
    d2.start()

    def step(j, carry):
        for b in range(NBUF):
            i = j * NBUF + b

            sb = (b - 1) % NBUF

            @pl.when((i - 1 >= 0) & (i - 1 < ni))
            def _():
                gdesc(sb).wait()
                sdesc(i - 1, sb).start()

            lb = (b + 1) % NBUF

            @pl.when(i + 1 < ni)
            def _():
                @pl.when(i + 1 - NBUF >= 0)
                def _():
                    sdesc(i + 1 - NBUF, lb).wait()
                e1, e2 = ldescs(i + 1, lb)
                e1.start()
                e2.start()

            @pl.when(i < ni)
            def _():
                f1, f2 = ldescs(i, b)
                f1.wait()
                f2.wait()
                gdesc(b).start(add=True)
        return carry

    lax.fori_loop(0, (ITERS + 1 + NBUF - 1) // NBUF, step, 0)

    for s in range(NBUF):
        sdesc(0, s).wait()


def kernel(x_features, up_x_features, unq_inv):
    idx = unq_inv.astype(jnp.int32)
    mesh = plsc.VectorSubcoreMesh(
        core_axis_name="c", subcore_axis_name="s",
        num_cores=NC, num_subcores=NS)
    f = pl.kernel(
        _sc_body,
        out_type=jax.ShapeDtypeStruct((M, C), jnp.float32),
        mesh=mesh,
        scratch_types=[
            pltpu.VMEM((NBUF, R), jnp.int32),
            pltpu.VMEM((NBUF, R, C), jnp.float32),
            pltpu.SemaphoreType.DMA((NBUF,)),
            pltpu.SemaphoreType.DMA((NBUF,)),
            pltpu.SemaphoreType.DMA((NBUF,)),
        ],
    )
    return f(x_features, up_x_features, idx)
